# Initial kernel scaffold; baseline (speedup 1.0000x reference)
import jax
import jax.numpy as jnp
from jax.experimental import pallas as pl


def kernel(emb_rel, relation_triplets, W_attn, b_attn, attn_vec, W_aggr, b_aggr):
    raise NotImplementedError("write your pallas kernel here")


# speedup vs baseline: 20.3385x; 20.3385x over previous
import jax
import jax.numpy as jnp
from jax import lax
from jax.experimental import pallas as pl
from jax.experimental.pallas import tpu as pltpu
from jax.experimental.pallas import tpu_sc as plsc

NUM_REL = 10000
E = 160000
DIN = 256
NH = 8
DH = 32

NC = 2
NS = 16
L = 16
HW = 128
TW = 256
CW = 144
C = 32
NCH = E // C
NP = NCH // 2
PPT = NP // NS
PREM = NP - PPT * NS
RPT = NUM_REL // NS
RB = 25
NRB = RPT // RB

_R = 1000


def _proj_kernel(e_ref, w_ref, b_ref, h_ref, t_ref):
    o = jnp.dot(e_ref[...], w_ref[...], preferred_element_type=jnp.float32)
    o = o + b_ref[...]
    h_ref[0] = o[:, 0:128]
    h_ref[1] = o[:, 128:256]
    t_ref[0] = o[:, 256:512]
    t_ref[1] = o[:, 512:768]


def _sc_kernel(h_tbl, t_tbl, idx2d, av_hbm, out_hbm,
               acc, ih, sidx, h0, h1, t0, t1, c0, c1, av_v,
               sem_i0, sem_i1, sem_h0, sem_h1, sem_t0, sem_t1,
               sem_s0, sem_s1):
    c = lax.axis_index("c")
    s = lax.axis_index("s")
    iota = lax.broadcasted_iota(jnp.int32, (L,), 0)
    zero16 = jnp.zeros((L,), jnp.float32)
    izero16 = jnp.zeros((L,), jnp.int32)
    row0 = s * RPT
    off = c * NUM_REL
    hb = [h0, h1]
    tb = [t0, t1]
    cb = [c0, c1]
    sem_i = [sem_i0, sem_i1]
    sem_h = [sem_h0, sem_h1]
    sem_t = [sem_t0, sem_t1]
    sem_s = [sem_s0, sem_s1]

    def zrow(r, carry):
        for j in range(CW // L):
            t0[r, pl.ds(j * L, L)] = zero16
        return carry
    lax.fori_loop(0, RB, zrow, 0)
    zsrc = t0.at[pl.ds(0, RB), pl.ds(0, CW)]
    for blk in range(NRB):
        pltpu.sync_copy(zsrc, acc.at[pl.ds(row0 + blk * RB, RB)])

    def zcon(r, carry):
        for j in range(CW // L):
            c0[r, pl.ds(j * L, L)] = zero16
            c1[r, pl.ds(j * L, L)] = zero16
        return carry
    lax.fori_loop(0, C, zcon, 0)
    for q in range(4):
        for k in range(C // L):
            sidx[q, pl.ds(k * L, L)] = izero16

    pltpu.sync_copy(av_hbm.at[pl.ds(c * HW, HW)], av_v)
    plsc.subcore_barrier()

    start_p = PPT * s + jnp.minimum(s, PREM)
    np_ = PPT + jnp.where(s < PREM, 1, 0)
    n0 = start_p * 2

    def adj_idx(slot, sslot):
        for k in range(C // L):
            sidx[sslot, pl.ds(k * L, L)] = ih[slot, pl.ds(k * L, L)]
        for k in range(2 * C // L):
            ih[slot, pl.ds(k * L, L)] = ih[slot, pl.ds(k * L, L)] + off

    def issue_gathers(slot):
        gh = pltpu.async_copy(h_tbl.at[ih.at[slot, pl.ds(0, C)]],
                              hb[slot], sem_h[slot])
        gt = pltpu.async_copy(t_tbl.at[ih.at[slot, pl.ds(C, C)]],
                              tb[slot], sem_t[slot])
        return gh, gt

    def wait_gathers(slot):
        pltpu.make_async_copy(h_tbl.at[ih.at[slot, pl.ds(0, C)]],
                              hb[slot], sem_h[slot]).wait()
        pltpu.make_async_copy(t_tbl.at[ih.at[slot, pl.ds(C, C)]],
                              tb[slot], sem_t[slot]).wait()

    def wait_scatter(slot):
        pltpu.make_async_copy(cb[slot], acc.at[sidx.at[0]],
                              sem_s[slot]).wait()

    pltpu.sync_copy(idx2d.at[n0], ih.at[0])
    adj_idx(0, n0 & 3)
    issue_gathers(0)
    pltpu.async_copy(idx2d.at[n0 + 1], ih.at[1], sem_i1)
    pltpu.async_copy(c0, acc.at[sidx.at[0]], sem_s0, add=True)
    pltpu.async_copy(c1, acc.at[sidx.at[0]], sem_s1, add=True)

    av_regs = [av_v[pl.ds(j * L, L)] for j in range(HW // L)]

    def compute_chunk(slot):
        hg, tg, cg = hb[slot], tb[slot], cb[slot]
        def edge_body(e, carry_e):
            ps = []
            for j in range(HW // L):
                gh = hg[e, pl.ds(j * L, L)]
                gt = tg[e, pl.ds(j * L, L)]
                ex = jnp.exp(gh + gt)
                avj = av_regs[j]
                ps.append((avj * ex - avj) / (ex + 1.0))
            den_vec = zero16
            ws = []
            for h in range(4):
                logit = jnp.sum(ps[2 * h] + ps[2 * h + 1])
                wv = jnp.exp(zero16 + logit)
                ws.append(wv)
                den_vec = jnp.where(iota == h, wv, den_vec)
            cg[e, pl.ds(HW, L)] = den_vec
            for j in range(HW // L):
                m = tg[e, pl.ds(HW + j * L, L)]
                cg[e, pl.ds(j * L, L)] = ws[j // 2] * m
            return carry_e
        lax.fori_loop(0, C, edge_body, 0, unroll=2)

    def pair_body(p, carry):
        n2 = p * 2
        for j in range(2):
            n = n2 + j
            g = j % 2
            gn = (j + 1) % 2
            pltpu.make_async_copy(idx2d.at[0], ih.at[gn], sem_i[gn]).wait()
            adj_idx(gn, (n + 1) & 3)
            issue_gathers(gn)
            wait_gathers(g)
            ch2 = jnp.minimum(n + 2, NCH - 1)
            pltpu.async_copy(idx2d.at[ch2], ih.at[g], sem_i[g])
            wait_scatter(g)
            compute_chunk(g)
            pltpu.async_copy(cb[g], acc.at[sidx.at[n & 3]], sem_s[g], add=True)
        return carry
    lax.fori_loop(start_p, start_p + np_, pair_body, 0)

    pltpu.make_async_copy(idx2d.at[0], ih.at[1], sem_i1).wait()
    wait_gathers(0)
    wait_scatter(0)
    wait_scatter(1)
    plsc.subcore_barrier()

    for blk in range(NRB):
        r0 = row0 + blk * RB
        pltpu.sync_copy(acc.at[pl.ds(r0, RB)], zsrc)
        def nrow(r, carry):
            rr = jnp.full((L,), 0, jnp.int32) + r
            for h in range(4):
                g = plsc.load_gather(t0, [rr, jnp.full((L,), HW + h, jnp.int32)])
                rec = 1.0 / (g + 1e-16)
                for j2 in range(2):
                    j = h * 2 + j2
                    h0[r, pl.ds(j * L, L)] = t0[r, pl.ds(j * L, L)] * rec
            return carry
        lax.fori_loop(0, RB, nrow, 0)
        pltpu.sync_copy(h0.at[pl.ds(0, RB)],
                        out_hbm.at[pl.ds(c * NUM_REL + r0, RB)])


def kernel(emb_rel, relation_triplets, W_attn, b_attn, attn_vec, W_aggr, b_aggr):
    Wh = W_attn[:, :DIN]
    Wt = W_attn[:, DIN:]
    Wcat = jnp.concatenate([
        2.0 * Wh[0:128], 2.0 * Wh[128:256],
        2.0 * Wt[0:128], W_aggr[0:128],
        2.0 * Wt[128:256], W_aggr[128:256],
    ], axis=0)
    z = jnp.zeros((128,), jnp.float32)
    bcat = jnp.concatenate([
        2.0 * b_attn[0:128], 2.0 * b_attn[128:256],
        z, b_aggr[0:128], z, b_aggr[128:256],
    ]).reshape(1, 768)

    h_t, t_t = pl.pallas_call(
        _proj_kernel,
        grid=(NUM_REL // _R,),
        in_specs=[
            pl.BlockSpec((_R, DIN), lambda i: (i, 0)),
            pl.BlockSpec((DIN, 768), lambda i: (0, 0)),
            pl.BlockSpec((1, 768), lambda i: (0, 0)),
        ],
        out_specs=[
            pl.BlockSpec((2, _R, HW), lambda i: (0, i, 0)),
            pl.BlockSpec((2, _R, TW), lambda i: (0, i, 0)),
        ],
        out_shape=[
            jax.ShapeDtypeStruct((2, NUM_REL, HW), jnp.float32),
            jax.ShapeDtypeStruct((2, NUM_REL, TW), jnp.float32),
        ],
    )(emb_rel, Wcat.T, bcat)

    h_tbl = h_t.reshape(NC * NUM_REL, HW)
    t_tbl = t_t.reshape(NC * NUM_REL, TW)
    head_idx = relation_triplets[:, 0]
    tail_idx = relation_triplets[:, 1]
    idx2d = jnp.concatenate(
        [head_idx.reshape(NCH, C), tail_idx.reshape(NCH, C)], axis=1)
    av_flat = attn_vec.reshape(-1)

    sck = pl.kernel(
        _sc_kernel,
        out_type=jax.ShapeDtypeStruct((NC * NUM_REL, HW), jnp.float32),
        mesh=plsc.VectorSubcoreMesh(core_axis_name="c", subcore_axis_name="s"),
        compiler_params=pltpu.CompilerParams(
            use_tc_tiling_on_sc=False, needs_layout_passes=False),
        scratch_types=[
            pltpu.VMEM_SHARED((NUM_REL, CW), jnp.float32),
            pltpu.VMEM((2, 2 * C), jnp.int32),
            pltpu.VMEM((4, C), jnp.int32),
            pltpu.VMEM((C, HW), jnp.float32),
            pltpu.VMEM((C, HW), jnp.float32),
            pltpu.VMEM((C, TW), jnp.float32),
            pltpu.VMEM((C, TW), jnp.float32),
            pltpu.VMEM((C, CW), jnp.float32),
            pltpu.VMEM((C, CW), jnp.float32),
            pltpu.VMEM((HW,), jnp.float32),
            pltpu.SemaphoreType.DMA,
            pltpu.SemaphoreType.DMA,
            pltpu.SemaphoreType.DMA,
            pltpu.SemaphoreType.DMA,
            pltpu.SemaphoreType.DMA,
            pltpu.SemaphoreType.DMA,
            pltpu.SemaphoreType.DMA,
            pltpu.SemaphoreType.DMA,
        ],
    )
    out = sck(h_tbl, t_tbl, idx2d, av_flat)
    return jnp.concatenate([out[:NUM_REL], out[NUM_REL:]], axis=1)
